# baseline (speedup 1.0000x reference)
import functools
import jax
import jax.numpy as jnp
from jax import lax
from jax.experimental import pallas as pl
from jax.experimental.pallas import tpu as pltpu
from jax.experimental.pallas import tpu_sc as plsc

VOCAB = 30522
HIDDEN = 768
MAX_POS = 512
TYPE_VOCAB = 16
BATCH = 128
SEQ = 512

NW = 32
TOK = BATCH * SEQ
TPW = TOK // NW
CH = 64
NCH = TPW // CH
INV_H = 1.0 / HIDDEN
EPS = 1e-12


def _sc_gather_body(ids_hbm, word_hbm, out_hbm,
                    idx0, idx1, buf0, buf1, gsem0, gsem1, osem0, osem1,
                    tpw=TPW, nch=NCH):
    wid = lax.axis_index("s") * 2 + lax.axis_index("c")
    base = wid * tpw
    idx = (idx0, idx1)
    buf = (buf0, buf1)
    gsem = (gsem0, gsem1)
    osem = (osem0, osem1)

    def fire(c, slot):
        pltpu.sync_copy(ids_hbm.at[pl.ds(base + c * CH, CH)], idx[slot])
        pltpu.async_copy(word_hbm.at[idx[slot]], buf[slot], gsem[slot])

    def wait_gather(slot):
        pltpu.make_async_copy(word_hbm.at[idx[slot]], buf[slot],
                              gsem[slot]).wait()

    def start_out(c, slot):
        pltpu.async_copy(buf[slot],
                         out_hbm.at[pl.ds(base + c * CH, CH), :], osem[slot])

    def wait_out(c, slot):
        pltpu.make_async_copy(buf[slot],
                              out_hbm.at[pl.ds(base + c * CH, CH), :],
                              osem[slot]).wait()

    fire(0, 0)
    for c in range(nch):
        slot = c % 2
        nslot = 1 - slot
        wait_gather(slot)
        if c + 1 < nch:
            if c >= 1:
                wait_out(c - 1, nslot)
            fire(c + 1, nslot)
        start_out(c, slot)
    wait_out(nch - 2, (nch - 2) % 2)
    wait_out(nch - 1, (nch - 1) % 2)


def _sc_gather(ids_flat, word_table):
    ntok = ids_flat.shape[0]
    tpw = ntok // NW
    mesh = plsc.VectorSubcoreMesh(core_axis_name="c", subcore_axis_name="s")
    f = pl.kernel(
        functools.partial(_sc_gather_body, tpw=tpw, nch=tpw // CH),
        out_type=jax.ShapeDtypeStruct((ntok, HIDDEN), jnp.float32),
        mesh=mesh,
        compiler_params=pltpu.CompilerParams(needs_layout_passes=False),
        scratch_types=[
            pltpu.VMEM((CH,), jnp.int32),
            pltpu.VMEM((CH,), jnp.int32),
            pltpu.VMEM((CH, HIDDEN), jnp.float32),
            pltpu.VMEM((CH, HIDDEN), jnp.float32),
            pltpu.SemaphoreType.DMA,
            pltpu.SemaphoreType.DMA,
            pltpu.SemaphoreType.DMA,
            pltpu.SemaphoreType.DMA,
        ],
    )
    return f(ids_flat, word_table)


def _tc_body(tt_ref, w_ref, pos_ref, type_ref, gam_ref, bet_ref, out_ref):
    w = w_ref[...]
    tt = tt_ref[0, 0, :]
    onehot = (tt[:, None] ==
              lax.broadcasted_iota(jnp.int32, (SEQ, TYPE_VOCAB), 1)
              ).astype(jnp.float32)
    ty = type_ref[...]
    ty_hi = ty.astype(jnp.bfloat16).astype(jnp.float32)
    temb = (jnp.dot(onehot, ty_hi, preferred_element_type=jnp.float32)
            + jnp.dot(onehot, ty - ty_hi,
                      preferred_element_type=jnp.float32))
    v = w + pos_ref[...] + temb
    mean = jnp.mean(v, axis=-1, keepdims=True)
    sq = jnp.mean(v * v, axis=-1, keepdims=True)
    rstd = lax.rsqrt(sq - mean * mean + EPS)
    out_ref[...] = (v - mean) * rstd * gam_ref[...] + bet_ref[...]


def _tc_stage(token_type_ids, gathered, pos_table, type_table, gamma, beta):
    nb = token_type_ids.shape[0]
    gamma2 = gamma.reshape(1, HIDDEN)
    beta2 = beta.reshape(1, HIDDEN)
    tt3 = token_type_ids.reshape(nb, 1, SEQ)
    out = pl.pallas_call(
        _tc_body,
        grid=(nb,),
        in_specs=[
            pl.BlockSpec((1, 1, SEQ), lambda b: (b, 0, 0)),
            pl.BlockSpec((SEQ, HIDDEN), lambda b: (b, 0)),
            pl.BlockSpec((MAX_POS, HIDDEN), lambda b: (0, 0)),
            pl.BlockSpec((TYPE_VOCAB, HIDDEN), lambda b: (0, 0)),
            pl.BlockSpec((1, HIDDEN), lambda b: (0, 0)),
            pl.BlockSpec((1, HIDDEN), lambda b: (0, 0)),
        ],
        out_specs=pl.BlockSpec((SEQ, HIDDEN), lambda b: (b, 0)),
        out_shape=jax.ShapeDtypeStruct((nb * SEQ, HIDDEN), jnp.float32),
    )(tt3, gathered, pos_table, type_table, gamma2, beta2)
    return out.reshape(nb, SEQ, HIDDEN)


NSPLIT = 1


@jax.jit
def _run(input_ids, token_type_ids, word_table, pos_table, type_table,
         gamma, beta):
    ids = input_ids.reshape(TOK)
    nb = BATCH // NSPLIT
    pieces = []
    gs = [_sc_gather(ids[i * nb * SEQ:(i + 1) * nb * SEQ], word_table)
          for i in range(NSPLIT)]
    for i in range(NSPLIT):
        pieces.append(_tc_stage(token_type_ids[i * nb:(i + 1) * nb], gs[i],
                                pos_table, type_table, gamma, beta))
    return jnp.concatenate(pieces, axis=0)


def kernel(input_ids, token_type_ids, word_table, pos_table, type_table,
           gamma, beta):
    return _run(input_ids.astype(jnp.int32), token_type_ids.astype(jnp.int32),
                word_table, pos_table, type_table, gamma, beta)


# speedup vs baseline: 1.1102x; 1.1102x over previous
import functools
import jax
import jax.numpy as jnp
from jax import lax
from jax.experimental import pallas as pl
from jax.experimental.pallas import tpu as pltpu
from jax.experimental.pallas import tpu_sc as plsc

VOCAB = 30522
HIDDEN = 768
MAX_POS = 512
TYPE_VOCAB = 16
BATCH = 128
SEQ = 512

NW = 32
TOK = BATCH * SEQ
TPW = TOK // NW
CH = 64
NCH = TPW // CH
INV_H = 1.0 / HIDDEN
EPS = 1e-12


def _sc_gather_body(ids_hbm, word_hbm, out_hbm,
                    idx0, idx1, buf0, buf1, gsem0, gsem1, osem0, osem1,
                    tpw=TPW, nch=NCH):
    wid = lax.axis_index("s") * 2 + lax.axis_index("c")
    base = wid * tpw
    idx = (idx0, idx1)
    buf = (buf0, buf1)
    gsem = (gsem0, gsem1)
    osem = (osem0, osem1)

    def fire(c, slot):
        pltpu.sync_copy(ids_hbm.at[pl.ds(base + c * CH, CH)], idx[slot])
        pltpu.async_copy(word_hbm.at[idx[slot]], buf[slot], gsem[slot])

    def wait_gather(slot):
        pltpu.make_async_copy(word_hbm.at[idx[slot]], buf[slot],
                              gsem[slot]).wait()

    def start_out(c, slot):
        pltpu.async_copy(buf[slot],
                         out_hbm.at[pl.ds(base + c * CH, CH), :], osem[slot])

    def wait_out(c, slot):
        pltpu.make_async_copy(buf[slot],
                              out_hbm.at[pl.ds(base + c * CH, CH), :],
                              osem[slot]).wait()

    fire(0, 0)
    for c in range(nch):
        slot = c % 2
        nslot = 1 - slot
        wait_gather(slot)
        if c + 1 < nch:
            if c >= 1:
                wait_out(c - 1, nslot)
            fire(c + 1, nslot)
        start_out(c, slot)
    wait_out(nch - 2, (nch - 2) % 2)
    wait_out(nch - 1, (nch - 1) % 2)


def _sc_gather(ids_flat, word_table):
    ntok = ids_flat.shape[0]
    tpw = ntok // NW
    mesh = plsc.VectorSubcoreMesh(core_axis_name="c", subcore_axis_name="s")
    f = pl.kernel(
        functools.partial(_sc_gather_body, tpw=tpw, nch=tpw // CH),
        out_type=jax.ShapeDtypeStruct((ntok, HIDDEN), jnp.float32),
        mesh=mesh,
        compiler_params=pltpu.CompilerParams(needs_layout_passes=False),
        scratch_types=[
            pltpu.VMEM((CH,), jnp.int32),
            pltpu.VMEM((CH,), jnp.int32),
            pltpu.VMEM((CH, HIDDEN), jnp.float32),
            pltpu.VMEM((CH, HIDDEN), jnp.float32),
            pltpu.SemaphoreType.DMA,
            pltpu.SemaphoreType.DMA,
            pltpu.SemaphoreType.DMA,
            pltpu.SemaphoreType.DMA,
        ],
    )
    return f(ids_flat, word_table)


BSEQ = 2
BTOK = BSEQ * SEQ


def _tc_body(tt_ref, w_ref, pos_ref, type_ref, gam_ref, bet_ref, out_ref):
    w = w_ref[...]
    tt = tt_ref[0].reshape(BTOK)
    onehot = (tt[:, None] ==
              lax.broadcasted_iota(jnp.int32, (BTOK, TYPE_VOCAB), 1)
              ).astype(jnp.float32)
    ty = type_ref[...]
    ty_hi = ty.astype(jnp.bfloat16).astype(jnp.float32)
    temb = (jnp.dot(onehot, ty_hi, preferred_element_type=jnp.float32)
            + jnp.dot(onehot, ty - ty_hi,
                      preferred_element_type=jnp.float32))
    v = w + pos_ref[...] + temb
    mean = jnp.mean(v, axis=-1, keepdims=True)
    sq = jnp.mean(v * v, axis=-1, keepdims=True)
    rstd = lax.rsqrt(sq - mean * mean + EPS)
    out_ref[...] = (v - mean) * rstd * gam_ref[...] + bet_ref[...]


def _tc_stage(token_type_ids, gathered, pos_table, type_table, gamma, beta):
    nb = token_type_ids.shape[0]
    nblk = nb // BSEQ
    gamma2 = gamma.reshape(1, HIDDEN)
    beta2 = beta.reshape(1, HIDDEN)
    tt3 = token_type_ids.reshape(nblk, 1, BTOK)
    pos2 = jnp.concatenate([pos_table] * BSEQ, axis=0)
    out = pl.pallas_call(
        _tc_body,
        grid=(nblk,),
        in_specs=[
            pl.BlockSpec((1, 1, BTOK), lambda b: (b, 0, 0)),
            pl.BlockSpec((BTOK, HIDDEN), lambda b: (b, 0)),
            pl.BlockSpec((BTOK, HIDDEN), lambda b: (0, 0)),
            pl.BlockSpec((TYPE_VOCAB, HIDDEN), lambda b: (0, 0)),
            pl.BlockSpec((1, HIDDEN), lambda b: (0, 0)),
            pl.BlockSpec((1, HIDDEN), lambda b: (0, 0)),
        ],
        out_specs=pl.BlockSpec((BTOK, HIDDEN), lambda b: (b, 0)),
        out_shape=jax.ShapeDtypeStruct((nb * SEQ, HIDDEN), jnp.float32),
    )(tt3, gathered, pos2, type_table, gamma2, beta2)
    return out.reshape(nb, SEQ, HIDDEN)


NSPLIT = 1


@jax.jit
def _run(input_ids, token_type_ids, word_table, pos_table, type_table,
         gamma, beta):
    ids = input_ids.reshape(TOK)
    nb = BATCH // NSPLIT
    pieces = []
    gs = [_sc_gather(ids[i * nb * SEQ:(i + 1) * nb * SEQ], word_table)
          for i in range(NSPLIT)]
    for i in range(NSPLIT):
        pieces.append(_tc_stage(token_type_ids[i * nb:(i + 1) * nb], gs[i],
                                pos_table, type_table, gamma, beta))
    return jnp.concatenate(pieces, axis=0)


def kernel(input_ids, token_type_ids, word_table, pos_table, type_table,
           gamma, beta):
    return _run(input_ids.astype(jnp.int32), token_type_ids.astype(jnp.int32),
                word_table, pos_table, type_table, gamma, beta)


# speedup vs baseline: 1.1760x; 1.0593x over previous
import functools
import jax
import jax.numpy as jnp
from jax import lax
from jax.experimental import pallas as pl
from jax.experimental.pallas import tpu as pltpu
from jax.experimental.pallas import tpu_sc as plsc

VOCAB = 30522
HIDDEN = 768
MAX_POS = 512
TYPE_VOCAB = 16
BATCH = 128
SEQ = 512

NW = 32
TOK = BATCH * SEQ
TPW = TOK // NW
CH = 64
NCH = TPW // CH
INV_H = 1.0 / HIDDEN
EPS = 1e-12


def _sc_gather_body(ids_hbm, word_hbm, out_hbm,
                    idx0, idx1, buf0, buf1, gsem0, gsem1, osem0, osem1,
                    tpw=TPW, nch=NCH):
    wid = lax.axis_index("s") * 2 + lax.axis_index("c")
    base = wid * tpw
    idx = (idx0, idx1)
    buf = (buf0, buf1)
    gsem = (gsem0, gsem1)
    osem = (osem0, osem1)

    def fire(c, slot):
        pltpu.sync_copy(ids_hbm.at[pl.ds(base + c * CH, CH)], idx[slot])
        pltpu.async_copy(word_hbm.at[idx[slot]], buf[slot], gsem[slot])

    def wait_gather(slot):
        pltpu.make_async_copy(word_hbm.at[idx[slot]], buf[slot],
                              gsem[slot]).wait()

    def start_out(c, slot):
        pltpu.async_copy(buf[slot],
                         out_hbm.at[pl.ds(base + c * CH, CH), :], osem[slot])

    def wait_out(c, slot):
        pltpu.make_async_copy(buf[slot],
                              out_hbm.at[pl.ds(base + c * CH, CH), :],
                              osem[slot]).wait()

    fire(0, 0)
    for c in range(nch):
        slot = c % 2
        nslot = 1 - slot
        wait_gather(slot)
        if c + 1 < nch:
            if c >= 1:
                wait_out(c - 1, nslot)
            fire(c + 1, nslot)
        start_out(c, slot)
    wait_out(nch - 2, (nch - 2) % 2)
    wait_out(nch - 1, (nch - 1) % 2)


def _sc_gather(ids_flat, word_table):
    ntok = ids_flat.shape[0]
    tpw = ntok // NW
    mesh = plsc.VectorSubcoreMesh(core_axis_name="c", subcore_axis_name="s")
    f = pl.kernel(
        functools.partial(_sc_gather_body, tpw=tpw, nch=tpw // CH),
        out_type=jax.ShapeDtypeStruct((ntok, HIDDEN), jnp.float32),
        mesh=mesh,
        compiler_params=pltpu.CompilerParams(needs_layout_passes=False),
        scratch_types=[
            pltpu.VMEM((CH,), jnp.int32),
            pltpu.VMEM((CH,), jnp.int32),
            pltpu.VMEM((CH, HIDDEN), jnp.float32),
            pltpu.VMEM((CH, HIDDEN), jnp.float32),
            pltpu.SemaphoreType.DMA,
            pltpu.SemaphoreType.DMA,
            pltpu.SemaphoreType.DMA,
            pltpu.SemaphoreType.DMA,
        ],
    )
    return f(ids_flat, word_table)


BSEQ = 4
BTOK = BSEQ * SEQ


def _tc_body(tt_ref, w_ref, pos_ref, type_ref, gam_ref, bet_ref, out_ref):
    w = w_ref[...]
    tt = tt_ref[0].reshape(BTOK)
    onehot = (tt[:, None] ==
              lax.broadcasted_iota(jnp.int32, (BTOK, TYPE_VOCAB), 1)
              ).astype(jnp.float32)
    ty = type_ref[...]
    ty_hi = ty.astype(jnp.bfloat16).astype(jnp.float32)
    temb = (jnp.dot(onehot, ty_hi, preferred_element_type=jnp.float32)
            + jnp.dot(onehot, ty - ty_hi,
                      preferred_element_type=jnp.float32))
    v = w + pos_ref[...] + temb
    mean = jnp.mean(v, axis=-1, keepdims=True)
    sq = jnp.mean(v * v, axis=-1, keepdims=True)
    rstd = lax.rsqrt(sq - mean * mean + EPS)
    out_ref[...] = (v - mean) * rstd * gam_ref[...] + bet_ref[...]


def _tc_stage(token_type_ids, gathered, pos_table, type_table, gamma, beta):
    nb = token_type_ids.shape[0]
    nblk = nb // BSEQ
    gamma2 = gamma.reshape(1, HIDDEN)
    beta2 = beta.reshape(1, HIDDEN)
    tt3 = token_type_ids.reshape(nblk, 1, BTOK)
    pos2 = jnp.concatenate([pos_table] * BSEQ, axis=0)
    out = pl.pallas_call(
        _tc_body,
        grid=(nblk,),
        in_specs=[
            pl.BlockSpec((1, 1, BTOK), lambda b: (b, 0, 0)),
            pl.BlockSpec((BTOK, HIDDEN), lambda b: (b, 0)),
            pl.BlockSpec((BTOK, HIDDEN), lambda b: (0, 0)),
            pl.BlockSpec((TYPE_VOCAB, HIDDEN), lambda b: (0, 0)),
            pl.BlockSpec((1, HIDDEN), lambda b: (0, 0)),
            pl.BlockSpec((1, HIDDEN), lambda b: (0, 0)),
        ],
        out_specs=pl.BlockSpec((BTOK, HIDDEN), lambda b: (b, 0)),
        out_shape=jax.ShapeDtypeStruct((nb * SEQ, HIDDEN), jnp.float32),
    )(tt3, gathered, pos2, type_table, gamma2, beta2)
    return out.reshape(nb, SEQ, HIDDEN)


NSPLIT = 1


@jax.jit
def _run(input_ids, token_type_ids, word_table, pos_table, type_table,
         gamma, beta):
    ids = input_ids.reshape(TOK)
    nb = BATCH // NSPLIT
    pieces = []
    gs = [_sc_gather(ids[i * nb * SEQ:(i + 1) * nb * SEQ], word_table)
          for i in range(NSPLIT)]
    for i in range(NSPLIT):
        pieces.append(_tc_stage(token_type_ids[i * nb:(i + 1) * nb], gs[i],
                                pos_table, type_table, gamma, beta))
    return jnp.concatenate(pieces, axis=0)


def kernel(input_ids, token_type_ids, word_table, pos_table, type_table,
           gamma, beta):
    return _run(input_ids.astype(jnp.int32), token_type_ids.astype(jnp.int32),
                word_table, pos_table, type_table, gamma, beta)
